# baseline (speedup 1.0000x reference)
import functools

import jax
import jax.numpy as jnp
from jax import lax
from jax.experimental import pallas as pl
from jax.experimental.pallas import tpu as pltpu
from jax.experimental.pallas import tpu_sc as plsc

_B, _S, _F = 1024, 50, 512
_N, _H, _C = 16, 128, 128
_CH = 128
_NCH = _B // _CH
_f32 = jnp.float32

_NW = 32
_RPW = _B // _NW


def _sc_pool(x_hbm, out_hbm, buf, orow, sem0, sem1, osem0, osem1):
    wid = lax.axis_index("s") * 2 + lax.axis_index("c")
    base = wid * _RPW

    def cp_in(i, slot, sem):
        return pltpu.make_async_copy(
            x_hbm.at[pl.ds(base + i, 1)], buf.at[pl.ds(slot, 1)], sem)

    def cp_out(i, slot, sem):
        return pltpu.make_async_copy(
            orow.at[pl.ds(slot, 1)], out_hbm.at[pl.ds(base + i, 1)], sem)

    cp_in(0, 0, sem0).start()
    cp_in(1, 1, sem1).start()

    def row_body(slot, sem, osem, i):
        cp_in(i, slot, sem).wait()

        @pl.when(i >= 2)
        def _():
            cp_out(i - 2, slot, osem).wait()

        def f_body(f, carry):
            acc = buf[slot, 0, pl.ds(f * 16, 16)]
            for s in range(1, _S):
                acc = acc + buf[slot, s, pl.ds(f * 16, 16)]
            orow[slot, pl.ds(f * 16, 16)] = acc
            return carry

        lax.fori_loop(0, _F // 16, f_body, 0)
        @pl.when(i + 2 < _RPW)
        def _():
            cp_in(i + 2, slot, sem).start()
        cp_out(i, slot, osem).start()

    def loop_body(j, carry):
        i = j * 2
        row_body(0, sem0, osem0, i)
        row_body(1, sem1, osem1, i + 1)
        return carry

    lax.fori_loop(0, _RPW // 2, loop_body, 0)
    cp_out(_RPW - 2, 0, osem0).wait()
    cp_out(_RPW - 1, 1, osem1).wait()


def _sc_pool_call(x):
    mesh = plsc.VectorSubcoreMesh(core_axis_name="c", subcore_axis_name="s")
    kern = functools.partial(
        pl.kernel,
        mesh=mesh,
        out_type=jax.ShapeDtypeStruct((_B, _F), _f32),
        scratch_types=[
            pltpu.VMEM((2, _S, _F), _f32),
            pltpu.VMEM((2, _F), _f32),
            pltpu.SemaphoreType.DMA,
            pltpu.SemaphoreType.DMA,
            pltpu.SemaphoreType.DMA,
            pltpu.SemaphoreType.DMA,
        ],
    )(_sc_pool)
    return kern(x)


def _topk3_mask(d2, prev_mask):
    dpad = jnp.concatenate(
        [d2, jnp.full((_CH, 128 - _N), jnp.inf, _f32)], axis=1)
    idxs = jax.lax.broadcasted_iota(jnp.int32, (_CH, 128), 1)
    active = jnp.zeros((_CH, 128), _f32)
    dsel = dpad
    for _ in range(3):
        mval = jnp.min(dsel, axis=1, keepdims=True)
        ismin = dsel == mval
        j = jnp.min(jnp.where(ismin, idxs, 128), axis=1, keepdims=True)
        sel = idxs == j
        active = jnp.where(sel, 1.0, active)
        dsel = jnp.where(sel, jnp.inf, dsel)
    return jnp.maximum(prev_mask, jnp.max(active, axis=0, keepdims=True))


def _mega(ps_hbm, eps_hbm, projW_ref, projb_ref, encW_ref, encb_ref,
          muW_ref, mub_ref, lvW_ref, lvb_ref, gcnW_ref, gcnb_ref,
          gateW_ref, gateb_ref, dummy_ref, out_ref,
          psbuf, epsbuf, zscr, pssem, epssem):

    def start_ps(c):
        pltpu.make_async_copy(
            ps_hbm.at[pl.ds(c * _CH, _CH)],
            psbuf.at[c % 2], pssem.at[c % 2]).start()

    def wait_ps(c):
        pltpu.make_async_copy(
            ps_hbm.at[pl.ds(c * _CH, _CH)],
            psbuf.at[c % 2], pssem.at[c % 2]).wait()

    for c in range(_NCH):
        pltpu.make_async_copy(
            eps_hbm.at[pl.ds(c * _CH, _CH)],
            epsbuf.at[pl.ds(c * _CH, _CH)],
            epssem.at[c]).start()
    start_ps(0)
    start_ps(1)

    encW3 = encW_ref[...]
    h0 = jnp.sum(encW3 * dummy_ref[...][None, :, :], axis=-1) + encb_ref[...]
    h0 = h0 * jax.nn.sigmoid(h0)
    muW3 = jnp.reshape(muW_ref[...], (_N, _C, _H))
    emb = jnp.sum(muW3 * h0[:, None, :], axis=-1) + mub_ref[...]

    encW2 = jnp.reshape(encW3, (_N * _H, _F))
    muW2 = muW_ref[...]
    lvW2 = lvW_ref[...]

    mask = jnp.zeros((1, 128), _f32)

    for c in range(_NCH):
        wait_ps(c)
        pooled = psbuf[c % 2] * (1.0 / _S)
        if c + 2 < _NCH:
            start_ps(c + 2)

        xp = jax.lax.dot_general(pooled, projW_ref[...],
                                 (((1,), (1,)), ((), ())),
                                 preferred_element_type=_f32) + projb_ref[...]
        diff = xp[:, None, :] - emb[None, :, :]
        d2 = jnp.sum(diff * diff, axis=-1)
        mask = _topk3_mask(d2, mask)

        hpre = jax.lax.dot_general(pooled, encW2, (((1,), (1,)), ((), ())),
                                   preferred_element_type=_f32)
        h = hpre + jnp.reshape(encb_ref[...], (1, _N * _H))
        h = h * jax.nn.sigmoid(h)

        pltpu.make_async_copy(eps_hbm.at[pl.ds(c * _CH, _CH)],
                              epsbuf.at[pl.ds(c * _CH, _CH)],
                              epssem.at[c]).wait()
        for n in range(_N):
            hn = h[:, n * _H:(n + 1) * _H]
            mu_n = jax.lax.dot_general(
                hn, muW2[n * _C:(n + 1) * _C, :], (((1,), (1,)), ((), ())),
                preferred_element_type=_f32) + mub_ref[n:n + 1, :]
            lv_n = jax.lax.dot_general(
                hn, lvW2[n * _C:(n + 1) * _C, :], (((1,), (1,)), ((), ())),
                preferred_element_type=_f32) + lvb_ref[n:n + 1, :]
            z_n = mu_n + epsbuf[pl.ds(c * _CH, _CH), n, :] * jnp.exp(0.5 * lv_n)
            zscr[pl.ds(c * _CH, _CH), pl.ds(n * _C, _C)] = z_n

    m = jnp.sum(mask)
    minv = 1.0 / jnp.maximum(m, 1.0)
    for c in range(_NCH):
        acc = jnp.zeros((_CH, _C), _f32)
        for n in range(_N):
            acc = acc + mask[0, n] * zscr[pl.ds(c * _CH, _CH),
                                          pl.ds(n * _C, _C)]
        zbar = acc * minv
        gcn = jax.lax.dot_general(zbar, gcnW_ref[...], (((1,), (1,)), ((), ())),
                                  preferred_element_type=_f32) \
            + gcnb_ref[...]
        thal = jnp.where(m == 0, jnp.zeros_like(zbar),
                         jnp.where(m <= 1, zbar, gcn))
        gate = jax.nn.sigmoid(
            jnp.sum(thal * gateW_ref[...], axis=1, keepdims=True)
            + gateb_ref[0])
        out_ref[pl.ds(c * _CH, _CH), :] = thal * gate


def kernel(x, proj_W, proj_b, enc_W, enc_b, mu_W, mu_b, lv_W, lv_b,
           gcn_W, gcn_b, gate_W, gate_b, dummy, eps):
    psum = _sc_pool_call(x)

    vmem = pl.BlockSpec(memory_space=pltpu.MemorySpace.VMEM)
    out = pl.pallas_call(
        _mega,
        in_specs=[
            pl.BlockSpec(memory_space=pltpu.MemorySpace.HBM),
            pl.BlockSpec(memory_space=pltpu.MemorySpace.HBM),
            vmem, vmem, vmem, vmem, vmem, vmem, vmem, vmem,
            vmem, vmem, vmem,
            pl.BlockSpec(memory_space=pltpu.MemorySpace.SMEM),
            vmem,
        ],
        out_specs=vmem,
        out_shape=jax.ShapeDtypeStruct((_B, _C), _f32),
        scratch_shapes=[
            pltpu.VMEM((2, _CH, _F), _f32),
            pltpu.VMEM((_B, _N, _C), _f32),
            pltpu.VMEM((_B, _N * _C), _f32),
            pltpu.SemaphoreType.DMA((2,)),
            pltpu.SemaphoreType.DMA((_NCH,)),
        ],
    )(psum, eps, proj_W, proj_b.reshape(1, _C), enc_W, enc_b,
      mu_W.reshape(_N * _C, _H), mu_b, lv_W.reshape(_N * _C, _H), lv_b,
      gcn_W, gcn_b.reshape(1, _C), gate_W, gate_b, dummy.reshape(1, _F))
    return out


# speedup vs baseline: 1.1738x; 1.1738x over previous
import functools

import jax
import jax.numpy as jnp
from jax import lax
from jax.experimental import pallas as pl
from jax.experimental.pallas import tpu as pltpu
from jax.experimental.pallas import tpu_sc as plsc

_B, _S, _F = 1024, 50, 512
_N, _H, _C = 16, 128, 128
_CH = 128
_NCH = _B // _CH
_f32 = jnp.float32

_SPLIT = 512
_NW = 32
_RPW = (_B - _SPLIT) // _NW
_NSUB = 4
_SUB = _CH // _NSUB


def _sc_pool(x_hbm, out_hbm, buf, orow, sem0, sem1, osem0, osem1):
    wid = lax.axis_index("s") * 2 + lax.axis_index("c")
    base_in = _SPLIT + wid * _RPW
    base_out = wid * _RPW

    def cp_in(i, slot, sem):
        return pltpu.make_async_copy(
            x_hbm.at[pl.ds(base_in + i, 1)], buf.at[pl.ds(slot, 1)], sem)

    def cp_out(i, slot, sem):
        return pltpu.make_async_copy(
            orow.at[pl.ds(slot, 1)], out_hbm.at[pl.ds(base_out + i, 1)], sem)

    cp_in(0, 0, sem0).start()
    cp_in(1, 1, sem1).start()

    def row_body(slot, sem, osem, i):
        cp_in(i, slot, sem).wait()

        @pl.when(i >= 2)
        def _():
            cp_out(i - 2, slot, osem).wait()

        def f_body(f, carry):
            acc = buf[slot, 0, pl.ds(f * 16, 16)]
            for s in range(1, _S):
                acc = acc + buf[slot, s, pl.ds(f * 16, 16)]
            orow[slot, pl.ds(f * 16, 16)] = acc
            return carry

        lax.fori_loop(0, _F // 16, f_body, 0)
        @pl.when(i + 2 < _RPW)
        def _():
            cp_in(i + 2, slot, sem).start()
        cp_out(i, slot, osem).start()

    def loop_body(j, carry):
        i = j * 2
        row_body(0, sem0, osem0, i)
        row_body(1, sem1, osem1, i + 1)
        return carry

    lax.fori_loop(0, _RPW // 2, loop_body, 0)
    cp_out(_RPW - 2, 0, osem0).wait()
    cp_out(_RPW - 1, 1, osem1).wait()


def _sc_pool_call(x):
    mesh = plsc.VectorSubcoreMesh(core_axis_name="c", subcore_axis_name="s")
    kern = functools.partial(
        pl.kernel,
        mesh=mesh,
        out_type=jax.ShapeDtypeStruct((_B - _SPLIT, _F), _f32),
        scratch_types=[
            pltpu.VMEM((2, _S, _F), _f32),
            pltpu.VMEM((2, _F), _f32),
            pltpu.SemaphoreType.DMA,
            pltpu.SemaphoreType.DMA,
            pltpu.SemaphoreType.DMA,
            pltpu.SemaphoreType.DMA,
        ],
    )(_sc_pool)
    return kern(x)


def _tc_pool(x_hbm, out_ref, xbuf, xsem):
    nch = _SPLIT // _CH

    def start_x(c):
        slot = c % 2
        for q in range(_NSUB):
            pltpu.make_async_copy(
                x_hbm.at[pl.ds(c * _CH + q * _SUB, _SUB)],
                xbuf.at[slot, pl.ds(q * _SUB, _SUB)],
                xsem.at[slot, q]).start()

    def wait_x(c):
        slot = c % 2
        for q in range(_NSUB):
            pltpu.make_async_copy(
                x_hbm.at[pl.ds(c * _CH + q * _SUB, _SUB)],
                xbuf.at[slot, pl.ds(q * _SUB, _SUB)],
                xsem.at[slot, q]).wait()

    start_x(0)
    start_x(1)
    for c in range(nch):
        wait_x(c)
        pooled = jnp.mean(xbuf[c % 2], axis=1)
        if c + 2 < nch:
            start_x(c + 2)
        out_ref[pl.ds(c * _CH, _CH), :] = pooled


def _topk3_mask(d2, prev_mask):
    dpad = jnp.concatenate(
        [d2, jnp.full((_CH, 128 - _N), jnp.inf, _f32)], axis=1)
    idxs = jax.lax.broadcasted_iota(jnp.int32, (_CH, 128), 1)
    active = jnp.zeros((_CH, 128), _f32)
    dsel = dpad
    for _ in range(3):
        mval = jnp.min(dsel, axis=1, keepdims=True)
        ismin = dsel == mval
        j = jnp.min(jnp.where(ismin, idxs, 128), axis=1, keepdims=True)
        sel = idxs == j
        active = jnp.where(sel, 1.0, active)
        dsel = jnp.where(sel, jnp.inf, dsel)
    return jnp.maximum(prev_mask, jnp.max(active, axis=0, keepdims=True))


def _mega(plo_ref, phi_hbm, eps_hbm, projW_ref, projb_ref, encW_ref,
          encb_ref, muW_ref, mub_ref, lvW_ref, lvb_ref, gcnW_ref, gcnb_ref,
          gateW_ref, gateb_ref, dummy_ref, out_ref,
          psbuf, epsbuf, zscr, pssem, epssem):
    nlo = _SPLIT // _CH

    def start_ps(c):
        pltpu.make_async_copy(
            phi_hbm.at[pl.ds((c - nlo) * _CH, _CH)],
            psbuf.at[c % 2], pssem.at[c % 2]).start()

    def wait_ps(c):
        pltpu.make_async_copy(
            phi_hbm.at[pl.ds((c - nlo) * _CH, _CH)],
            psbuf.at[c % 2], pssem.at[c % 2]).wait()

    for c in range(_NCH):
        pltpu.make_async_copy(
            eps_hbm.at[pl.ds(c * _CH, _CH)],
            epsbuf.at[pl.ds(c * _CH, _CH)],
            epssem.at[c]).start()
    start_ps(nlo)
    start_ps(nlo + 1)

    encW3 = encW_ref[...]
    h0 = jnp.sum(encW3 * dummy_ref[...][None, :, :], axis=-1) + encb_ref[...]
    h0 = h0 * jax.nn.sigmoid(h0)
    muW3 = jnp.reshape(muW_ref[...], (_N, _C, _H))
    emb = jnp.sum(muW3 * h0[:, None, :], axis=-1) + mub_ref[...]

    encW2 = jnp.reshape(encW3, (_N * _H, _F))
    muW2 = muW_ref[...]
    lvW2 = lvW_ref[...]

    mask = jnp.zeros((1, 128), _f32)

    for c in range(_NCH):
        if c < nlo:
            pooled = plo_ref[pl.ds(c * _CH, _CH), :]
        else:
            wait_ps(c)
            pooled = psbuf[c % 2] * (1.0 / _S)
            if c + 2 < _NCH:
                start_ps(c + 2)

        xp = jax.lax.dot_general(pooled, projW_ref[...],
                                 (((1,), (1,)), ((), ())),
                                 preferred_element_type=_f32) + projb_ref[...]
        diff = xp[:, None, :] - emb[None, :, :]
        d2 = jnp.sum(diff * diff, axis=-1)
        mask = _topk3_mask(d2, mask)

        hpre = jax.lax.dot_general(pooled, encW2, (((1,), (1,)), ((), ())),
                                   preferred_element_type=_f32)
        h = hpre + jnp.reshape(encb_ref[...], (1, _N * _H))
        h = h * jax.nn.sigmoid(h)

        pltpu.make_async_copy(eps_hbm.at[pl.ds(c * _CH, _CH)],
                              epsbuf.at[pl.ds(c * _CH, _CH)],
                              epssem.at[c]).wait()
        for n in range(_N):
            hn = h[:, n * _H:(n + 1) * _H]
            mu_n = jax.lax.dot_general(
                hn, muW2[n * _C:(n + 1) * _C, :], (((1,), (1,)), ((), ())),
                preferred_element_type=_f32) + mub_ref[n:n + 1, :]
            lv_n = jax.lax.dot_general(
                hn, lvW2[n * _C:(n + 1) * _C, :], (((1,), (1,)), ((), ())),
                preferred_element_type=_f32) + lvb_ref[n:n + 1, :]
            z_n = mu_n + epsbuf[pl.ds(c * _CH, _CH), n, :] * jnp.exp(0.5 * lv_n)
            zscr[pl.ds(c * _CH, _CH), pl.ds(n * _C, _C)] = z_n

    m = jnp.sum(mask)
    minv = 1.0 / jnp.maximum(m, 1.0)
    for c in range(_NCH):
        acc = jnp.zeros((_CH, _C), _f32)
        for n in range(_N):
            acc = acc + mask[0, n] * zscr[pl.ds(c * _CH, _CH),
                                          pl.ds(n * _C, _C)]
        zbar = acc * minv
        gcn = jax.lax.dot_general(zbar, gcnW_ref[...], (((1,), (1,)), ((), ())),
                                  preferred_element_type=_f32) \
            + gcnb_ref[...]
        thal = jnp.where(m == 0, jnp.zeros_like(zbar),
                         jnp.where(m <= 1, zbar, gcn))
        gate = jax.nn.sigmoid(
            jnp.sum(thal * gateW_ref[...], axis=1, keepdims=True)
            + gateb_ref[0])
        out_ref[pl.ds(c * _CH, _CH), :] = thal * gate


def kernel(x, proj_W, proj_b, enc_W, enc_b, mu_W, mu_b, lv_W, lv_b,
           gcn_W, gcn_b, gate_W, gate_b, dummy, eps):
    psum_hi = _sc_pool_call(x)

    pool_lo = pl.pallas_call(
        _tc_pool,
        in_specs=[pl.BlockSpec(memory_space=pltpu.MemorySpace.HBM)],
        out_specs=pl.BlockSpec(memory_space=pltpu.MemorySpace.VMEM),
        out_shape=jax.ShapeDtypeStruct((_SPLIT, _F), _f32),
        scratch_shapes=[
            pltpu.VMEM((2, _CH, _S, _F), _f32),
            pltpu.SemaphoreType.DMA((2, _NSUB)),
        ],
    )(x)

    vmem = pl.BlockSpec(memory_space=pltpu.MemorySpace.VMEM)
    out = pl.pallas_call(
        _mega,
        in_specs=[
            vmem,
            pl.BlockSpec(memory_space=pltpu.MemorySpace.HBM),
            pl.BlockSpec(memory_space=pltpu.MemorySpace.HBM),
            vmem, vmem, vmem, vmem, vmem, vmem, vmem, vmem,
            vmem, vmem, vmem,
            pl.BlockSpec(memory_space=pltpu.MemorySpace.SMEM),
            vmem,
        ],
        out_specs=vmem,
        out_shape=jax.ShapeDtypeStruct((_B, _C), _f32),
        scratch_shapes=[
            pltpu.VMEM((2, _CH, _F), _f32),
            pltpu.VMEM((_B, _N, _C), _f32),
            pltpu.VMEM((_B, _N * _C), _f32),
            pltpu.SemaphoreType.DMA((2,)),
            pltpu.SemaphoreType.DMA((_NCH,)),
        ],
    )(pool_lo, psum_hi, eps, proj_W, proj_b.reshape(1, _C), enc_W, enc_b,
      mu_W.reshape(_N * _C, _H), mu_b, lv_W.reshape(_N * _C, _H), lv_b,
      gcn_W, gcn_b.reshape(1, _C), gate_W, gate_b, dummy.reshape(1, _F))
    return out
